# Initial kernel scaffold; baseline (speedup 1.0000x reference)
import jax
import jax.numpy as jnp
from jax.experimental import pallas as pl


def kernel(node_features, edge_index, W1, b1, W2, b2):
    raise NotImplementedError("write your pallas kernel here")


# speedup vs baseline: 7.6078x; 7.6078x over previous
import jax
import jax.numpy as jnp
from jax import lax
from jax.experimental import pallas as pl
from jax.experimental.pallas import tpu as pltpu
from jax.experimental.pallas import tpu_sc as plsc

N = 10000
D = 128
NP = 10240
NC = 2
NS = 16
NW = NC * NS
CH = 128
EPW = 10240
NCH = EPW // CH
EPAD = NW * EPW
SENT = N + 64
RPS = NP // NS

_MESH = plsc.VectorSubcoreMesh(
    core_axis_name="c", subcore_axis_name="s", num_cores=NC, num_subcores=NS)


L = 16


def _sc_deg_body(dst_hbm, zeros_hbm, out_hbm, dst_v, accv):
    c = lax.axis_index("c")
    s = lax.axis_index("s")
    w = c * NS + s
    pltpu.sync_copy(dst_hbm.at[w], dst_v)
    pltpu.sync_copy(zeros_hbm, accv)
    ones = jnp.full((L,), 1.0, jnp.float32)

    def step(t, carry):
        plsc.addupdate_scatter(accv, [dst_v[pl.ds(t * L, L)]], ones)
        return carry

    lax.fori_loop(0, EPW // L, step, 0)
    pltpu.sync_copy(accv, out_hbm.at[pl.ds(w * NP, NP)])


_sc_deg = pl.kernel(
    _sc_deg_body,
    out_type=jax.ShapeDtypeStruct((NW * NP,), jnp.float32),
    mesh=_MESH,
    scratch_types=[
        pltpu.VMEM((EPW,), jnp.int32),
        pltpu.VMEM((NP,), jnp.float32),
    ],
    compiler_params=pltpu.CompilerParams(needs_layout_passes=False),
)


def _sc_agg_body(tab_hbm, src_hbm, dst_hbm, zeros_hbm, out_hbm,
                 src_v, dst_v, gbuf, acc, sg, ss):
    c = lax.axis_index("c")
    s = lax.axis_index("s")
    w = c * NS + s
    pltpu.sync_copy(src_hbm.at[w], src_v)
    pltpu.sync_copy(dst_hbm.at[w], dst_v)
    pltpu.sync_copy(zeros_hbm.at[pl.ds(s * RPS, RPS)], acc.at[pl.ds(s * RPS, RPS)])
    plsc.subcore_barrier()

    def chunk(j, carry):
        pltpu.async_copy(tab_hbm.at[src_v.at[j]], gbuf, sg).wait()
        pltpu.async_copy(gbuf, acc.at[dst_v.at[j]], ss, add=True).wait()
        return carry

    lax.fori_loop(0, NCH, chunk, 0)
    plsc.subcore_barrier()
    pltpu.sync_copy(acc.at[pl.ds(s * RPS, RPS)], out_hbm.at[c, pl.ds(s * RPS, RPS)])


_sc_agg = pl.kernel(
    _sc_agg_body,
    out_type=jax.ShapeDtypeStruct((NC, NP, D), jnp.float32),
    mesh=_MESH,
    scratch_types=[
        pltpu.VMEM((NCH, CH), jnp.int32),
        pltpu.VMEM((NCH, CH), jnp.int32),
        pltpu.VMEM((CH, D), jnp.float32),
        pltpu.VMEM_SHARED((NP, D), jnp.float32),
        pltpu.SemaphoreType.DMA,
        pltpu.SemaphoreType.DMA,
    ],
)


BLK = 512
NBLK = NP // BLK


def _tc1_body(x_ref, w1_ref, degp_ref, z_ref, p_ref, dinv_ref):
    deg = jnp.sum(degp_ref[...], axis=0)
    dinv = lax.rsqrt(deg + 1.0)
    z = jnp.dot(x_ref[...], w1_ref[...],
                preferred_element_type=jnp.float32, precision=lax.Precision.HIGHEST)
    z_ref[...] = z
    p_ref[...] = z * dinv
    dinv_ref[...] = dinv


_tc1 = pl.pallas_call(
    _tc1_body,
    grid=(NBLK,),
    in_specs=[
        pl.BlockSpec((BLK, D), lambda i: (i, 0)),
        pl.BlockSpec((D, D), lambda i: (0, 0)),
        pl.BlockSpec((NW, BLK, 1), lambda i: (0, i, 0)),
    ],
    out_specs=[
        pl.BlockSpec((BLK, D), lambda i: (i, 0)),
        pl.BlockSpec((BLK, D), lambda i: (i, 0)),
        pl.BlockSpec((BLK, 1), lambda i: (i, 0)),
    ],
    out_shape=[
        jax.ShapeDtypeStruct((NP, D), jnp.float32),
        jax.ShapeDtypeStruct((NP, D), jnp.float32),
        jax.ShapeDtypeStruct((NP, 1), jnp.float32),
    ],
)


def _tc2_body(sp_ref, z1_ref, dinv_ref, b1_ref, w2_ref, z2_ref, p2_ref):
    i = pl.program_id(0)
    sp = sp_ref[...]
    dinv = dinv_ref[...]
    h = dinv * (sp[0] + sp[1]) + dinv * dinv * z1_ref[...] + b1_ref[...]
    h = jnp.maximum(h, 0.0)
    rows = i * BLK + lax.broadcasted_iota(jnp.int32, (BLK, 1), 0)
    h = jnp.where(rows < N, h, 0.0)
    z2 = jnp.dot(h, w2_ref[...],
                 preferred_element_type=jnp.float32, precision=lax.Precision.HIGHEST)
    z2_ref[...] = z2
    p2_ref[...] = z2 * dinv


_tc2 = pl.pallas_call(
    _tc2_body,
    grid=(NBLK,),
    in_specs=[
        pl.BlockSpec((2, BLK, D), lambda i: (0, i, 0)),
        pl.BlockSpec((BLK, D), lambda i: (i, 0)),
        pl.BlockSpec((BLK, 1), lambda i: (i, 0)),
        pl.BlockSpec((1, D), lambda i: (0, 0)),
        pl.BlockSpec((D, D), lambda i: (0, 0)),
    ],
    out_specs=[
        pl.BlockSpec((BLK, D), lambda i: (i, 0)),
        pl.BlockSpec((BLK, D), lambda i: (i, 0)),
    ],
    out_shape=[
        jax.ShapeDtypeStruct((NP, D), jnp.float32),
        jax.ShapeDtypeStruct((NP, D), jnp.float32),
    ],
)


def _tc3_body(sp_ref, z2_ref, dinv_ref, b2_ref, h2_ref, ss_ref):
    i = pl.program_id(0)
    sp = sp_ref[...]
    dinv = dinv_ref[...]
    h = dinv * (sp[0] + sp[1]) + dinv * dinv * z2_ref[...] + b2_ref[...]
    rows = i * BLK + lax.broadcasted_iota(jnp.int32, (BLK, 1), 0)
    h = jnp.where(rows < N, h, 0.0)
    h2_ref[...] = h

    @pl.when(i == 0)
    def _():
        ss_ref[...] = jnp.zeros_like(ss_ref)

    ss_ref[...] = ss_ref[...] + jnp.sum(h * h, axis=0, keepdims=True)


_tc3 = pl.pallas_call(
    _tc3_body,
    grid=(NBLK,),
    in_specs=[
        pl.BlockSpec((2, BLK, D), lambda i: (0, i, 0)),
        pl.BlockSpec((BLK, D), lambda i: (i, 0)),
        pl.BlockSpec((BLK, 1), lambda i: (i, 0)),
        pl.BlockSpec((1, D), lambda i: (0, 0)),
    ],
    out_specs=[
        pl.BlockSpec((BLK, D), lambda i: (i, 0)),
        pl.BlockSpec((1, D), lambda i: (0, 0)),
    ],
    out_shape=[
        jax.ShapeDtypeStruct((NP, D), jnp.float32),
        jax.ShapeDtypeStruct((1, D), jnp.float32),
    ],
)


def _tc4_body(h2_ref, ss_ref, out_ref):
    denom = jnp.maximum(jnp.sqrt(ss_ref[...]), 1e-12)
    out_ref[...] = h2_ref[...] / denom


_tc4 = pl.pallas_call(
    _tc4_body,
    grid=(NBLK,),
    in_specs=[
        pl.BlockSpec((BLK, D), lambda i: (i, 0)),
        pl.BlockSpec((1, D), lambda i: (0, 0)),
    ],
    out_specs=pl.BlockSpec((BLK, D), lambda i: (i, 0)),
    out_shape=jax.ShapeDtypeStruct((NP, D), jnp.float32),
)


def kernel(node_features, edge_index, W1, b1, W2, b2):
    x = jnp.zeros((NP, D), jnp.float32).at[:N].set(node_features)
    ei = edge_index.astype(jnp.int32)
    pad = jnp.full((EPAD - ei.shape[1],), SENT, jnp.int32)
    src = jnp.concatenate([ei[0], pad]).reshape(NW, NCH, CH)
    dst = jnp.concatenate([ei[1], pad]).reshape(NW, NCH, CH)
    zeros_deg = jnp.zeros((NP,), jnp.float32)
    zeros_tab = jnp.zeros((NP, D), jnp.float32)

    degp = _sc_deg(dst.reshape(NW, EPW), zeros_deg)
    z1, p1, dinv = _tc1(x, W1, degp.reshape(NW, NP, 1))
    s1 = _sc_agg(p1, src, dst, zeros_tab)
    z2, p2 = _tc2(s1, z1, dinv, b1.reshape(1, D), W2)
    s2 = _sc_agg(p2, src, dst, zeros_tab)
    h2, ss = _tc3(s2, z2, dinv, b2.reshape(1, D))
    out = _tc4(h2, ss)
    return out[:N]


# speedup vs baseline: 16.6605x; 2.1899x over previous
import jax
import jax.numpy as jnp
from jax import lax
from jax.experimental import pallas as pl
from jax.experimental.pallas import tpu as pltpu
from jax.experimental.pallas import tpu_sc as plsc

N = 10000
D = 128
NP = 10240
NC = 2
NS = 16
NW = NC * NS
CH = 128
EPW = 10240
NCH = EPW // CH
EPAD = NW * EPW
SENT = N + 64
RPS = NP // NS

_MESH = plsc.VectorSubcoreMesh(
    core_axis_name="c", subcore_axis_name="s", num_cores=NC, num_subcores=NS)


L = 16


def _sc_deg_body(dst_hbm, zeros_hbm, out_hbm, dst_v, accv):
    c = lax.axis_index("c")
    s = lax.axis_index("s")
    w = c * NS + s
    pltpu.sync_copy(dst_hbm.at[w], dst_v)
    pltpu.sync_copy(zeros_hbm, accv)
    ones = jnp.full((L,), 1.0, jnp.float32)

    def step(t, carry):
        plsc.addupdate_scatter(accv, [dst_v[pl.ds(t * L, L)]], ones)
        return carry

    lax.fori_loop(0, EPW // L, step, 0)
    pltpu.sync_copy(accv, out_hbm.at[pl.ds(w * NP, NP)])


_sc_deg = pl.kernel(
    _sc_deg_body,
    out_type=jax.ShapeDtypeStruct((NW * NP,), jnp.float32),
    mesh=_MESH,
    scratch_types=[
        pltpu.VMEM((EPW,), jnp.int32),
        pltpu.VMEM((NP,), jnp.float32),
    ],
    compiler_params=pltpu.CompilerParams(needs_layout_passes=False),
)


NBUF = 2
NIB = 4


def _sc_agg_body(tab_hbm, idx_hbm, zeros_hbm, out_hbm, idx_v, gb, acc, si, sg, ss):
    c = lax.axis_index("c")
    s = lax.axis_index("s")
    w = c * NS + s
    pltpu.sync_copy(zeros_hbm.at[pl.ds(s * RPS, RPS)], acc.at[pl.ds(s * RPS, RPS)])
    for q in range(NBUF):
        pltpu.async_copy(idx_hbm.at[w, q], idx_v.at[q], si.at[q])
    plsc.subcore_barrier()

    @pl.loop(0, NCH, step=NBUF)
    def _(g):
        for b in range(NBUF):
            j = g + b
            q = j % NIB

            @pl.when(j >= NBUF)
            def _():
                qq = (j - NBUF) % NIB
                pltpu.make_async_copy(gb.at[b], acc.at[idx_v.at[qq, 1]],
                                      ss.at[b]).wait()

            @pl.when(j + NBUF < NCH)
            def _():
                qn = (j + NBUF) % NIB
                pltpu.async_copy(idx_hbm.at[w, j + NBUF], idx_v.at[qn], si.at[qn])

            pltpu.make_async_copy(idx_hbm.at[w, j], idx_v.at[q], si.at[q]).wait()
            pltpu.async_copy(tab_hbm.at[idx_v.at[q, 0]], gb.at[b], sg.at[b])
        for b in range(NBUF):
            j = g + b
            q = j % NIB
            pltpu.make_async_copy(tab_hbm.at[idx_v.at[q, 0]], gb.at[b],
                                  sg.at[b]).wait()
            pltpu.async_copy(gb.at[b], acc.at[idx_v.at[q, 1]], ss.at[b], add=True)

    for b in range(NBUF):
        j = NCH - NBUF + b
        pltpu.make_async_copy(gb.at[b], acc.at[idx_v.at[j % NIB, 1]],
                              ss.at[b]).wait()

    plsc.subcore_barrier()
    pltpu.sync_copy(acc.at[pl.ds(s * RPS, RPS)], out_hbm.at[c, pl.ds(s * RPS, RPS)])


_sc_agg = pl.kernel(
    _sc_agg_body,
    out_type=jax.ShapeDtypeStruct((NC, NP, D), jnp.float32),
    mesh=_MESH,
    scratch_types=[
        pltpu.VMEM((NIB, 2, CH), jnp.int32),
        pltpu.VMEM((NBUF, CH, D), jnp.float32),
        pltpu.VMEM_SHARED((NP, D), jnp.float32),
        pltpu.SemaphoreType.DMA((NIB,)),
        pltpu.SemaphoreType.DMA((NBUF,)),
        pltpu.SemaphoreType.DMA((NBUF,)),
    ],
)


BLK = 512
NBLK = NP // BLK


def _tc1_body(x_ref, w1_ref, degp_ref, z_ref, p_ref, dinv_ref):
    deg = jnp.sum(degp_ref[...], axis=0)
    dinv = lax.rsqrt(deg + 1.0)
    z = jnp.dot(x_ref[...], w1_ref[...],
                preferred_element_type=jnp.float32, precision=lax.Precision.HIGHEST)
    z_ref[...] = z
    p_ref[...] = z * dinv
    dinv_ref[...] = dinv


_tc1 = pl.pallas_call(
    _tc1_body,
    grid=(NBLK,),
    in_specs=[
        pl.BlockSpec((BLK, D), lambda i: (i, 0)),
        pl.BlockSpec((D, D), lambda i: (0, 0)),
        pl.BlockSpec((NW, BLK, 1), lambda i: (0, i, 0)),
    ],
    out_specs=[
        pl.BlockSpec((BLK, D), lambda i: (i, 0)),
        pl.BlockSpec((BLK, D), lambda i: (i, 0)),
        pl.BlockSpec((BLK, 1), lambda i: (i, 0)),
    ],
    out_shape=[
        jax.ShapeDtypeStruct((NP, D), jnp.float32),
        jax.ShapeDtypeStruct((NP, D), jnp.float32),
        jax.ShapeDtypeStruct((NP, 1), jnp.float32),
    ],
)


def _tc2_body(sp_ref, z1_ref, dinv_ref, b1_ref, w2_ref, z2_ref, p2_ref):
    i = pl.program_id(0)
    sp = sp_ref[...]
    dinv = dinv_ref[...]
    h = dinv * (sp[0] + sp[1]) + dinv * dinv * z1_ref[...] + b1_ref[...]
    h = jnp.maximum(h, 0.0)
    rows = i * BLK + lax.broadcasted_iota(jnp.int32, (BLK, 1), 0)
    h = jnp.where(rows < N, h, 0.0)
    z2 = jnp.dot(h, w2_ref[...],
                 preferred_element_type=jnp.float32, precision=lax.Precision.HIGHEST)
    z2_ref[...] = z2
    p2_ref[...] = z2 * dinv


_tc2 = pl.pallas_call(
    _tc2_body,
    grid=(NBLK,),
    in_specs=[
        pl.BlockSpec((2, BLK, D), lambda i: (0, i, 0)),
        pl.BlockSpec((BLK, D), lambda i: (i, 0)),
        pl.BlockSpec((BLK, 1), lambda i: (i, 0)),
        pl.BlockSpec((1, D), lambda i: (0, 0)),
        pl.BlockSpec((D, D), lambda i: (0, 0)),
    ],
    out_specs=[
        pl.BlockSpec((BLK, D), lambda i: (i, 0)),
        pl.BlockSpec((BLK, D), lambda i: (i, 0)),
    ],
    out_shape=[
        jax.ShapeDtypeStruct((NP, D), jnp.float32),
        jax.ShapeDtypeStruct((NP, D), jnp.float32),
    ],
)


def _tc3_body(sp_ref, z2_ref, dinv_ref, b2_ref, h2_ref, ss_ref):
    i = pl.program_id(0)
    sp = sp_ref[...]
    dinv = dinv_ref[...]
    h = dinv * (sp[0] + sp[1]) + dinv * dinv * z2_ref[...] + b2_ref[...]
    rows = i * BLK + lax.broadcasted_iota(jnp.int32, (BLK, 1), 0)
    h = jnp.where(rows < N, h, 0.0)
    h2_ref[...] = h

    @pl.when(i == 0)
    def _():
        ss_ref[...] = jnp.zeros_like(ss_ref)

    ss_ref[...] = ss_ref[...] + jnp.sum(h * h, axis=0, keepdims=True)


_tc3 = pl.pallas_call(
    _tc3_body,
    grid=(NBLK,),
    in_specs=[
        pl.BlockSpec((2, BLK, D), lambda i: (0, i, 0)),
        pl.BlockSpec((BLK, D), lambda i: (i, 0)),
        pl.BlockSpec((BLK, 1), lambda i: (i, 0)),
        pl.BlockSpec((1, D), lambda i: (0, 0)),
    ],
    out_specs=[
        pl.BlockSpec((BLK, D), lambda i: (i, 0)),
        pl.BlockSpec((1, D), lambda i: (0, 0)),
    ],
    out_shape=[
        jax.ShapeDtypeStruct((NP, D), jnp.float32),
        jax.ShapeDtypeStruct((1, D), jnp.float32),
    ],
)


def _tc4_body(h2_ref, ss_ref, out_ref):
    denom = jnp.maximum(jnp.sqrt(ss_ref[...]), 1e-12)
    out_ref[...] = h2_ref[...] / denom


_tc4 = pl.pallas_call(
    _tc4_body,
    grid=(NBLK,),
    in_specs=[
        pl.BlockSpec((BLK, D), lambda i: (i, 0)),
        pl.BlockSpec((1, D), lambda i: (0, 0)),
    ],
    out_specs=pl.BlockSpec((BLK, D), lambda i: (i, 0)),
    out_shape=jax.ShapeDtypeStruct((NP, D), jnp.float32),
)


def kernel(node_features, edge_index, W1, b1, W2, b2):
    x = jnp.zeros((NP, D), jnp.float32).at[:N].set(node_features)
    ei = edge_index.astype(jnp.int32)
    pad = N + jnp.arange(EPAD - ei.shape[1], dtype=jnp.int32) % (NP - N)
    src_f = jnp.concatenate([ei[0], pad])
    dst_f = jnp.concatenate([ei[1], pad])
    idx = jnp.stack([src_f.reshape(NW, NCH, CH),
                     dst_f.reshape(NW, NCH, CH)], axis=2)
    zeros_deg = jnp.zeros((NP,), jnp.float32)
    zeros_tab = jnp.zeros((NP, D), jnp.float32)

    degp = _sc_deg(dst_f.reshape(NW, EPW), zeros_deg)
    z1, p1, dinv = _tc1(x, W1, degp.reshape(NW, NP, 1))
    s1 = _sc_agg(p1, idx, zeros_tab)
    z2, p2 = _tc2(s1, z1, dinv, b1.reshape(1, D), W2)
    s2 = _sc_agg(p2, idx, zeros_tab)
    h2, ss = _tc3(s2, z2, dinv, b2.reshape(1, D))
    out = _tc4(h2, ss)
    return out[:N]


# speedup vs baseline: 24.2840x; 1.4576x over previous
import jax
import jax.numpy as jnp
from jax import lax
from jax.experimental import pallas as pl
from jax.experimental.pallas import tpu as pltpu
from jax.experimental.pallas import tpu_sc as plsc

N = 10000
D = 128
NP = 10240
NC = 2
NS = 16
NW = NC * NS
CH = 128
EPW = 10240
NCH = EPW // CH
EPAD = NW * EPW
SENT = N + 64
RPS = NP // NS

_MESH = plsc.VectorSubcoreMesh(
    core_axis_name="c", subcore_axis_name="s", num_cores=NC, num_subcores=NS)


L = 16


def _sc_deg_body(dst_hbm, zeros_hbm, out_hbm, dst_v, accv):
    c = lax.axis_index("c")
    s = lax.axis_index("s")
    w = c * NS + s
    pltpu.sync_copy(dst_hbm.at[w], dst_v)
    pltpu.sync_copy(zeros_hbm, accv)
    ones = jnp.full((L,), 1.0, jnp.float32)

    def step(t, carry):
        plsc.addupdate_scatter(accv, [dst_v[pl.ds(t * L, L)]], ones)
        return carry

    lax.fori_loop(0, EPW // L, step, 0)
    pltpu.sync_copy(accv, out_hbm.at[pl.ds(w * NP, NP)])


_sc_deg = pl.kernel(
    _sc_deg_body,
    out_type=jax.ShapeDtypeStruct((NW * NP,), jnp.float32),
    mesh=_MESH,
    scratch_types=[
        pltpu.VMEM((EPW,), jnp.int32),
        pltpu.VMEM((NP,), jnp.float32),
    ],
    compiler_params=pltpu.CompilerParams(needs_layout_passes=False),
)


NBUF = 2
NIB = 4


def _sc_agg_body(tab_hbm, idx_hbm, zeros_hbm, out_hbm, idx_v, gb, acc, si, sg, ss):
    c = lax.axis_index("c")
    s = lax.axis_index("s")
    w = c * NS + s
    pltpu.sync_copy(zeros_hbm.at[pl.ds(s * RPS, RPS)], acc.at[pl.ds(s * RPS, RPS)])
    for q in range(NBUF):
        pltpu.async_copy(idx_hbm.at[w, q], idx_v.at[q], si.at[q])
    plsc.subcore_barrier()

    @pl.loop(0, NCH, step=NBUF)
    def _(g):
        for b in range(NBUF):
            j = g + b
            q = j % NIB

            @pl.when(j >= NBUF)
            def _():
                qq = (j - NBUF) % NIB
                pltpu.make_async_copy(gb.at[b], acc.at[idx_v.at[qq, 1]],
                                      ss.at[b]).wait()

            @pl.when(j + NBUF < NCH)
            def _():
                qn = (j + NBUF) % NIB
                pltpu.async_copy(idx_hbm.at[w, j + NBUF], idx_v.at[qn], si.at[qn])

            pltpu.make_async_copy(idx_hbm.at[w, j], idx_v.at[q], si.at[q]).wait()
            pltpu.async_copy(tab_hbm.at[idx_v.at[q, 0]], gb.at[b], sg.at[b])
        for b in range(NBUF):
            j = g + b
            q = j % NIB
            pltpu.make_async_copy(tab_hbm.at[idx_v.at[q, 0]], gb.at[b],
                                  sg.at[b]).wait()
            pltpu.async_copy(gb.at[b], acc.at[idx_v.at[q, 1]], ss.at[b], add=True)

    for b in range(NBUF):
        j = NCH - NBUF + b
        pltpu.make_async_copy(gb.at[b], acc.at[idx_v.at[j % NIB, 1]],
                              ss.at[b]).wait()

    plsc.subcore_barrier()
    pltpu.sync_copy(acc.at[pl.ds(s * RPS, RPS)], out_hbm.at[c, pl.ds(s * RPS, RPS)])


_sc_agg = pl.kernel(
    _sc_agg_body,
    out_type=jax.ShapeDtypeStruct((NC, NP, D), jnp.float32),
    mesh=_MESH,
    scratch_types=[
        pltpu.VMEM((NIB, 2, CH), jnp.int32),
        pltpu.VMEM((NBUF, CH, D), jnp.float32),
        pltpu.VMEM_SHARED((NP, D), jnp.float32),
        pltpu.SemaphoreType.DMA((NIB,)),
        pltpu.SemaphoreType.DMA((NBUF,)),
        pltpu.SemaphoreType.DMA((NBUF,)),
    ],
)


BLK = 512
NBLK = NP // BLK


def _tc0_body(degp_ref, dinv_ref):
    dinv_ref[...] = lax.rsqrt(jnp.sum(degp_ref[...], axis=0) + 1.0)


_tc0 = pl.pallas_call(
    _tc0_body,
    grid=(1,),
    in_specs=[pl.BlockSpec((NW, NP // D, D), lambda i: (0, 0, 0))],
    out_specs=pl.BlockSpec((NP // D, D), lambda i: (0, 0)),
    out_shape=jax.ShapeDtypeStruct((NP // D, D), jnp.float32),
)


def _tc1_body(x_ref, w1_ref, dinv_ref, z_ref, p_ref):
    dinv = dinv_ref[...]
    z = jnp.dot(x_ref[...], w1_ref[...],
                preferred_element_type=jnp.float32, precision=lax.Precision.HIGHEST)
    z_ref[...] = z
    p_ref[...] = z * dinv


_tc1 = pl.pallas_call(
    _tc1_body,
    grid=(NBLK,),
    in_specs=[
        pl.BlockSpec((BLK, D), lambda i: (i, 0)),
        pl.BlockSpec((D, D), lambda i: (0, 0)),
        pl.BlockSpec((BLK, 1), lambda i: (i, 0)),
    ],
    out_specs=[
        pl.BlockSpec((BLK, D), lambda i: (i, 0)),
        pl.BlockSpec((BLK, D), lambda i: (i, 0)),
    ],
    out_shape=[
        jax.ShapeDtypeStruct((NP, D), jnp.float32),
        jax.ShapeDtypeStruct((NP, D), jnp.float32),
    ],
)


def _tc2_body(sp_ref, z1_ref, dinv_ref, b1_ref, w2_ref, z2_ref, p2_ref):
    i = pl.program_id(0)
    sp = sp_ref[...]
    dinv = dinv_ref[...]
    h = dinv * (sp[0] + sp[1]) + dinv * dinv * z1_ref[...] + b1_ref[...]
    h = jnp.maximum(h, 0.0)
    rows = i * BLK + lax.broadcasted_iota(jnp.int32, (BLK, 1), 0)
    h = jnp.where(rows < N, h, 0.0)
    z2 = jnp.dot(h, w2_ref[...],
                 preferred_element_type=jnp.float32, precision=lax.Precision.HIGHEST)
    z2_ref[...] = z2
    p2_ref[...] = z2 * dinv


_tc2 = pl.pallas_call(
    _tc2_body,
    grid=(NBLK,),
    in_specs=[
        pl.BlockSpec((2, BLK, D), lambda i: (0, i, 0)),
        pl.BlockSpec((BLK, D), lambda i: (i, 0)),
        pl.BlockSpec((BLK, 1), lambda i: (i, 0)),
        pl.BlockSpec((1, D), lambda i: (0, 0)),
        pl.BlockSpec((D, D), lambda i: (0, 0)),
    ],
    out_specs=[
        pl.BlockSpec((BLK, D), lambda i: (i, 0)),
        pl.BlockSpec((BLK, D), lambda i: (i, 0)),
    ],
    out_shape=[
        jax.ShapeDtypeStruct((NP, D), jnp.float32),
        jax.ShapeDtypeStruct((NP, D), jnp.float32),
    ],
)


def _tc3_body(sp_ref, z2_ref, dinv_ref, b2_ref, h2_ref, ss_ref):
    i = pl.program_id(0)
    sp = sp_ref[...]
    dinv = dinv_ref[...]
    h = dinv * (sp[0] + sp[1]) + dinv * dinv * z2_ref[...] + b2_ref[...]
    rows = i * BLK + lax.broadcasted_iota(jnp.int32, (BLK, 1), 0)
    h = jnp.where(rows < N, h, 0.0)
    h2_ref[...] = h

    @pl.when(i == 0)
    def _():
        ss_ref[...] = jnp.zeros_like(ss_ref)

    ss_ref[...] = ss_ref[...] + jnp.sum(h * h, axis=0, keepdims=True)


_tc3 = pl.pallas_call(
    _tc3_body,
    grid=(NBLK,),
    in_specs=[
        pl.BlockSpec((2, BLK, D), lambda i: (0, i, 0)),
        pl.BlockSpec((BLK, D), lambda i: (i, 0)),
        pl.BlockSpec((BLK, 1), lambda i: (i, 0)),
        pl.BlockSpec((1, D), lambda i: (0, 0)),
    ],
    out_specs=[
        pl.BlockSpec((BLK, D), lambda i: (i, 0)),
        pl.BlockSpec((1, D), lambda i: (0, 0)),
    ],
    out_shape=[
        jax.ShapeDtypeStruct((NP, D), jnp.float32),
        jax.ShapeDtypeStruct((1, D), jnp.float32),
    ],
)


def _tc4_body(h2_ref, ss_ref, out_ref):
    denom = jnp.maximum(jnp.sqrt(ss_ref[...]), 1e-12)
    out_ref[...] = h2_ref[...] / denom


_tc4 = pl.pallas_call(
    _tc4_body,
    grid=(NBLK,),
    in_specs=[
        pl.BlockSpec((BLK, D), lambda i: (i, 0)),
        pl.BlockSpec((1, D), lambda i: (0, 0)),
    ],
    out_specs=pl.BlockSpec((BLK, D), lambda i: (i, 0)),
    out_shape=jax.ShapeDtypeStruct((NP, D), jnp.float32),
)


def kernel(node_features, edge_index, W1, b1, W2, b2):
    x = jnp.zeros((NP, D), jnp.float32).at[:N].set(node_features)
    ei = edge_index.astype(jnp.int32)
    pad = N + jnp.arange(EPAD - ei.shape[1], dtype=jnp.int32) % (NP - N)
    src_f = jnp.concatenate([ei[0], pad])
    dst_f = jnp.concatenate([ei[1], pad])
    idx = jnp.stack([src_f.reshape(NW, NCH, CH),
                     dst_f.reshape(NW, NCH, CH)], axis=2)
    zeros_deg = jnp.zeros((NP,), jnp.float32)
    zeros_tab = jnp.zeros((NP, D), jnp.float32)

    degp = _sc_deg(dst_f.reshape(NW, EPW), zeros_deg)
    dinv = _tc0(degp.reshape(NW, NP // D, D)).reshape(NP, 1)
    z1, p1 = _tc1(x, W1, dinv)
    s1 = _sc_agg(p1, idx, zeros_tab)
    z2, p2 = _tc2(s1, z1, dinv, b1.reshape(1, D), W2)
    s2 = _sc_agg(p2, idx, zeros_tab)
    h2, ss = _tc3(s2, z2, dinv, b2.reshape(1, D))
    out = _tc4(h2, ss)
    return out[:N]


# speedup vs baseline: 27.9573x; 1.1513x over previous
import jax
import jax.numpy as jnp
from jax import lax
from jax.experimental import pallas as pl
from jax.experimental.pallas import tpu as pltpu
from jax.experimental.pallas import tpu_sc as plsc

N = 10000
D = 128
NP = 10240
NC = 2
NS = 16
NW = NC * NS
CH = 64
EPW = 10240
NCH = EPW // CH
EPAD = NW * EPW
SENT = N + 64
RPS = NP // NS

_MESH = plsc.VectorSubcoreMesh(
    core_axis_name="c", subcore_axis_name="s", num_cores=NC, num_subcores=NS)


L = 16


def _sc_deg_body(dst_hbm, zeros_hbm, out_hbm, dst_v, accv):
    c = lax.axis_index("c")
    s = lax.axis_index("s")
    w = c * NS + s
    pltpu.sync_copy(dst_hbm.at[w], dst_v)
    pltpu.sync_copy(zeros_hbm, accv)
    ones = jnp.full((L,), 1.0, jnp.float32)

    def step(t, carry):
        plsc.addupdate_scatter(accv, [dst_v[pl.ds(t * L, L)]], ones)
        return carry

    lax.fori_loop(0, EPW // L, step, 0)
    pltpu.sync_copy(accv, out_hbm.at[pl.ds(w * NP, NP)])


_sc_deg = pl.kernel(
    _sc_deg_body,
    out_type=jax.ShapeDtypeStruct((NW * NP,), jnp.float32),
    mesh=_MESH,
    scratch_types=[
        pltpu.VMEM((EPW,), jnp.int32),
        pltpu.VMEM((NP,), jnp.float32),
    ],
    compiler_params=pltpu.CompilerParams(needs_layout_passes=False),
)


NBUF = 4
NIB = 8


def _sc_agg_body(tab_hbm, idx_hbm, zeros_hbm, out_hbm, idx_v, gb, acc, si, sg, ss):
    c = lax.axis_index("c")
    s = lax.axis_index("s")
    w = c * NS + s
    pltpu.sync_copy(zeros_hbm.at[pl.ds(s * RPS, RPS)], acc.at[pl.ds(s * RPS, RPS)])
    for q in range(NBUF):
        pltpu.async_copy(idx_hbm.at[w, q], idx_v.at[q], si.at[q])
    plsc.subcore_barrier()

    @pl.loop(0, NCH, step=NBUF)
    def _(g):
        for b in range(NBUF):
            j = g + b
            q = j % NIB

            @pl.when(j >= NBUF)
            def _():
                qq = (j - NBUF) % NIB
                pltpu.make_async_copy(gb.at[b], acc.at[idx_v.at[qq, 1]],
                                      ss.at[b]).wait()

            @pl.when(j + NBUF < NCH)
            def _():
                qn = (j + NBUF) % NIB
                pltpu.async_copy(idx_hbm.at[w, j + NBUF], idx_v.at[qn], si.at[qn])

            pltpu.make_async_copy(idx_hbm.at[w, j], idx_v.at[q], si.at[q]).wait()
            pltpu.async_copy(tab_hbm.at[idx_v.at[q, 0]], gb.at[b], sg.at[b])
        for b in range(NBUF):
            j = g + b
            q = j % NIB
            pltpu.make_async_copy(tab_hbm.at[idx_v.at[q, 0]], gb.at[b],
                                  sg.at[b]).wait()
            pltpu.async_copy(gb.at[b], acc.at[idx_v.at[q, 1]], ss.at[b], add=True)

    for b in range(NBUF):
        j = NCH - NBUF + b
        pltpu.make_async_copy(gb.at[b], acc.at[idx_v.at[j % NIB, 1]],
                              ss.at[b]).wait()

    plsc.subcore_barrier()
    pltpu.sync_copy(acc.at[pl.ds(s * RPS, RPS)], out_hbm.at[c, pl.ds(s * RPS, RPS)])


_sc_agg = pl.kernel(
    _sc_agg_body,
    out_type=jax.ShapeDtypeStruct((NC, NP, D), jnp.float32),
    mesh=_MESH,
    scratch_types=[
        pltpu.VMEM((NIB, 2, CH), jnp.int32),
        pltpu.VMEM((NBUF, CH, D), jnp.float32),
        pltpu.VMEM_SHARED((NP, D), jnp.float32),
        pltpu.SemaphoreType.DMA((NIB,)),
        pltpu.SemaphoreType.DMA((NBUF,)),
        pltpu.SemaphoreType.DMA((NBUF,)),
    ],
)


BLK = 512
NBLK = NP // BLK


def _tc0_body(degp_ref, dinv_ref):
    dinv_ref[...] = lax.rsqrt(jnp.sum(degp_ref[...], axis=0) + 1.0)


_tc0 = pl.pallas_call(
    _tc0_body,
    grid=(1,),
    in_specs=[pl.BlockSpec((NW, NP // D, D), lambda i: (0, 0, 0))],
    out_specs=pl.BlockSpec((NP // D, D), lambda i: (0, 0)),
    out_shape=jax.ShapeDtypeStruct((NP // D, D), jnp.float32),
)


def _tc1_body(x_ref, w1_ref, dinv_ref, z_ref, p_ref):
    dinv = dinv_ref[...]
    z = jnp.dot(x_ref[...], w1_ref[...],
                preferred_element_type=jnp.float32, precision=lax.Precision.HIGHEST)
    z_ref[...] = z
    p_ref[...] = z * dinv


_tc1 = pl.pallas_call(
    _tc1_body,
    grid=(NBLK,),
    in_specs=[
        pl.BlockSpec((BLK, D), lambda i: (i, 0)),
        pl.BlockSpec((D, D), lambda i: (0, 0)),
        pl.BlockSpec((BLK, 1), lambda i: (i, 0)),
    ],
    out_specs=[
        pl.BlockSpec((BLK, D), lambda i: (i, 0)),
        pl.BlockSpec((BLK, D), lambda i: (i, 0)),
    ],
    out_shape=[
        jax.ShapeDtypeStruct((NP, D), jnp.float32),
        jax.ShapeDtypeStruct((NP, D), jnp.float32),
    ],
)


def _tc2_body(sp_ref, z1_ref, dinv_ref, b1_ref, w2_ref, z2_ref, p2_ref):
    i = pl.program_id(0)
    sp = sp_ref[...]
    dinv = dinv_ref[...]
    h = dinv * (sp[0] + sp[1]) + dinv * dinv * z1_ref[...] + b1_ref[...]
    h = jnp.maximum(h, 0.0)
    rows = i * BLK + lax.broadcasted_iota(jnp.int32, (BLK, 1), 0)
    h = jnp.where(rows < N, h, 0.0)
    z2 = jnp.dot(h, w2_ref[...],
                 preferred_element_type=jnp.float32, precision=lax.Precision.HIGHEST)
    z2_ref[...] = z2
    p2_ref[...] = z2 * dinv


_tc2 = pl.pallas_call(
    _tc2_body,
    grid=(NBLK,),
    in_specs=[
        pl.BlockSpec((2, BLK, D), lambda i: (0, i, 0)),
        pl.BlockSpec((BLK, D), lambda i: (i, 0)),
        pl.BlockSpec((BLK, 1), lambda i: (i, 0)),
        pl.BlockSpec((1, D), lambda i: (0, 0)),
        pl.BlockSpec((D, D), lambda i: (0, 0)),
    ],
    out_specs=[
        pl.BlockSpec((BLK, D), lambda i: (i, 0)),
        pl.BlockSpec((BLK, D), lambda i: (i, 0)),
    ],
    out_shape=[
        jax.ShapeDtypeStruct((NP, D), jnp.float32),
        jax.ShapeDtypeStruct((NP, D), jnp.float32),
    ],
)


def _tc3_body(sp_ref, z2_ref, dinv_ref, b2_ref, h2_ref, ss_ref):
    i = pl.program_id(0)
    sp = sp_ref[...]
    dinv = dinv_ref[...]
    h = dinv * (sp[0] + sp[1]) + dinv * dinv * z2_ref[...] + b2_ref[...]
    rows = i * BLK + lax.broadcasted_iota(jnp.int32, (BLK, 1), 0)
    h = jnp.where(rows < N, h, 0.0)
    h2_ref[...] = h

    @pl.when(i == 0)
    def _():
        ss_ref[...] = jnp.zeros_like(ss_ref)

    ss_ref[...] = ss_ref[...] + jnp.sum(h * h, axis=0, keepdims=True)


_tc3 = pl.pallas_call(
    _tc3_body,
    grid=(NBLK,),
    in_specs=[
        pl.BlockSpec((2, BLK, D), lambda i: (0, i, 0)),
        pl.BlockSpec((BLK, D), lambda i: (i, 0)),
        pl.BlockSpec((BLK, 1), lambda i: (i, 0)),
        pl.BlockSpec((1, D), lambda i: (0, 0)),
    ],
    out_specs=[
        pl.BlockSpec((BLK, D), lambda i: (i, 0)),
        pl.BlockSpec((1, D), lambda i: (0, 0)),
    ],
    out_shape=[
        jax.ShapeDtypeStruct((NP, D), jnp.float32),
        jax.ShapeDtypeStruct((1, D), jnp.float32),
    ],
)


def _tc4_body(h2_ref, ss_ref, out_ref):
    denom = jnp.maximum(jnp.sqrt(ss_ref[...]), 1e-12)
    out_ref[...] = h2_ref[...] / denom


_tc4 = pl.pallas_call(
    _tc4_body,
    grid=(NBLK,),
    in_specs=[
        pl.BlockSpec((BLK, D), lambda i: (i, 0)),
        pl.BlockSpec((1, D), lambda i: (0, 0)),
    ],
    out_specs=pl.BlockSpec((BLK, D), lambda i: (i, 0)),
    out_shape=jax.ShapeDtypeStruct((NP, D), jnp.float32),
)


def kernel(node_features, edge_index, W1, b1, W2, b2):
    x = jnp.zeros((NP, D), jnp.float32).at[:N].set(node_features)
    ei = edge_index.astype(jnp.int32)
    pad = N + jnp.arange(EPAD - ei.shape[1], dtype=jnp.int32) % (NP - N)
    src_f = jnp.concatenate([ei[0], pad])
    dst_f = jnp.concatenate([ei[1], pad])
    idx = jnp.stack([src_f.reshape(NW, NCH, CH),
                     dst_f.reshape(NW, NCH, CH)], axis=2)
    zeros_deg = jnp.zeros((NP,), jnp.float32)
    zeros_tab = jnp.zeros((NP, D), jnp.float32)

    degp = _sc_deg(dst_f.reshape(NW, EPW), zeros_deg)
    dinv = _tc0(degp.reshape(NW, NP // D, D)).reshape(NP, 1)
    z1, p1 = _tc1(x, W1, dinv)
    s1 = _sc_agg(p1, idx, zeros_tab)
    z2, p2 = _tc2(s1, z1, dinv, b1.reshape(1, D), W2)
    s2 = _sc_agg(p2, idx, zeros_tab)
    h2, ss = _tc3(s2, z2, dinv, b2.reshape(1, D))
    out = _tc4(h2, ss)
    return out[:N]


# speedup vs baseline: 28.4632x; 1.0181x over previous
import jax
import jax.numpy as jnp
from jax import lax
from jax.experimental import pallas as pl
from jax.experimental.pallas import tpu as pltpu
from jax.experimental.pallas import tpu_sc as plsc

N = 10000
D = 128
NP = 10240
NC = 2
NS = 16
NW = NC * NS
CH = 64
EPW = 10240
NCH = EPW // CH
EPAD = NW * EPW
SENT = N + 64
RPS = NP // NS

_MESH = plsc.VectorSubcoreMesh(
    core_axis_name="c", subcore_axis_name="s", num_cores=NC, num_subcores=NS)


L = 16


def _sc_deg_body(dst_hbm, zeros_hbm, out_hbm, dst_v, accv):
    c = lax.axis_index("c")
    s = lax.axis_index("s")
    w = c * NS + s
    pltpu.sync_copy(dst_hbm.at[w], dst_v)
    pltpu.sync_copy(zeros_hbm, accv)
    ones = jnp.full((L,), 1.0, jnp.float32)

    def step(t, carry):
        plsc.addupdate_scatter(accv, [dst_v[pl.ds(t * L, L)]], ones)
        return carry

    lax.fori_loop(0, EPW // L, step, 0)
    pltpu.sync_copy(accv, out_hbm.at[pl.ds(w * NP, NP)])


_sc_deg = pl.kernel(
    _sc_deg_body,
    out_type=jax.ShapeDtypeStruct((NW * NP,), jnp.float32),
    mesh=_MESH,
    scratch_types=[
        pltpu.VMEM((EPW,), jnp.int32),
        pltpu.VMEM((NP,), jnp.float32),
    ],
    compiler_params=pltpu.CompilerParams(needs_layout_passes=False),
)


NBUF = 4
NIB = 8


def _sc_agg_body(tab_hbm, idx_hbm, zeros_hbm, out_hbm, idx_v, gb, acc, si, sg, ss):
    c = lax.axis_index("c")
    s = lax.axis_index("s")
    w = c * NS + s
    pltpu.sync_copy(zeros_hbm.at[pl.ds(s * RPS, RPS)], acc.at[pl.ds(s * RPS, RPS)])
    for q in range(NBUF):
        pltpu.async_copy(idx_hbm.at[w, q], idx_v.at[q], si.at[q])
    plsc.subcore_barrier()

    @pl.loop(0, NCH, step=NBUF)
    def _(g):
        for b in range(NBUF):
            j = g + b
            q = j % NIB

            @pl.when(j >= NBUF)
            def _():
                qq = (j - NBUF) % NIB
                pltpu.make_async_copy(gb.at[b], acc.at[idx_v.at[qq, 1]],
                                      ss.at[b]).wait()

            @pl.when(j + NBUF < NCH)
            def _():
                qn = (j + NBUF) % NIB
                pltpu.async_copy(idx_hbm.at[w, j + NBUF], idx_v.at[qn], si.at[qn])

            pltpu.make_async_copy(idx_hbm.at[w, j], idx_v.at[q], si.at[q]).wait()
            pltpu.async_copy(tab_hbm.at[idx_v.at[q, 0]], gb.at[b], sg.at[b])
        for b in range(NBUF):
            j = g + b
            q = j % NIB
            pltpu.make_async_copy(tab_hbm.at[idx_v.at[q, 0]], gb.at[b],
                                  sg.at[b]).wait()
            pltpu.async_copy(gb.at[b], acc.at[idx_v.at[q, 1]], ss.at[b], add=True)

    for b in range(NBUF):
        j = NCH - NBUF + b
        pltpu.make_async_copy(gb.at[b], acc.at[idx_v.at[j % NIB, 1]],
                              ss.at[b]).wait()

    plsc.subcore_barrier()
    pltpu.sync_copy(acc.at[pl.ds(s * RPS, RPS)], out_hbm.at[c, pl.ds(s * RPS, RPS)])


_sc_agg = pl.kernel(
    _sc_agg_body,
    out_type=jax.ShapeDtypeStruct((NC, NP, D), jnp.float32),
    mesh=_MESH,
    scratch_types=[
        pltpu.VMEM((NIB, 2, CH), jnp.int32),
        pltpu.VMEM((NBUF, CH, D), jnp.float32),
        pltpu.VMEM_SHARED((NP, D), jnp.float32),
        pltpu.SemaphoreType.DMA((NIB,)),
        pltpu.SemaphoreType.DMA((NBUF,)),
        pltpu.SemaphoreType.DMA((NBUF,)),
    ],
)


BLK = 512
NBLK = NP // BLK


def _tc0_body(degp_ref, dinv_ref):
    dinv_ref[...] = lax.rsqrt(jnp.sum(degp_ref[...], axis=0) + 1.0)


_tc0 = pl.pallas_call(
    _tc0_body,
    grid=(1,),
    in_specs=[pl.BlockSpec((NW, NP // D, D), lambda i: (0, 0, 0))],
    out_specs=pl.BlockSpec((NP // D, D), lambda i: (0, 0)),
    out_shape=jax.ShapeDtypeStruct((NP // D, D), jnp.float32),
)


def _tc1_body(x_ref, w1_ref, dinv_ref, z_ref, p_ref):
    dinv = dinv_ref[...]
    z = jnp.dot(x_ref[...], w1_ref[...],
                preferred_element_type=jnp.float32, precision=lax.Precision.HIGHEST)
    z_ref[...] = z
    p_ref[...] = z * dinv


_tc1 = pl.pallas_call(
    _tc1_body,
    grid=(NBLK,),
    in_specs=[
        pl.BlockSpec((BLK, D), lambda i: (i, 0)),
        pl.BlockSpec((D, D), lambda i: (0, 0)),
        pl.BlockSpec((BLK, 1), lambda i: (i, 0)),
    ],
    out_specs=[
        pl.BlockSpec((BLK, D), lambda i: (i, 0)),
        pl.BlockSpec((BLK, D), lambda i: (i, 0)),
    ],
    out_shape=[
        jax.ShapeDtypeStruct((NP, D), jnp.float32),
        jax.ShapeDtypeStruct((NP, D), jnp.float32),
    ],
)


def _tc2_body(sp_ref, z1_ref, dinv_ref, b1_ref, w2_ref, z2_ref, p2_ref):
    i = pl.program_id(0)
    sp = sp_ref[...]
    dinv = dinv_ref[...]
    h = dinv * (sp[0] + sp[1]) + dinv * dinv * z1_ref[...] + b1_ref[...]
    h = jnp.maximum(h, 0.0)
    rows = i * BLK + lax.broadcasted_iota(jnp.int32, (BLK, 1), 0)
    h = jnp.where(rows < N, h, 0.0)
    z2 = jnp.dot(h, w2_ref[...],
                 preferred_element_type=jnp.float32, precision=lax.Precision.HIGHEST)
    z2_ref[...] = z2
    p2_ref[...] = z2 * dinv


_tc2 = pl.pallas_call(
    _tc2_body,
    grid=(NBLK,),
    in_specs=[
        pl.BlockSpec((2, BLK, D), lambda i: (0, i, 0)),
        pl.BlockSpec((BLK, D), lambda i: (i, 0)),
        pl.BlockSpec((BLK, 1), lambda i: (i, 0)),
        pl.BlockSpec((1, D), lambda i: (0, 0)),
        pl.BlockSpec((D, D), lambda i: (0, 0)),
    ],
    out_specs=[
        pl.BlockSpec((BLK, D), lambda i: (i, 0)),
        pl.BlockSpec((BLK, D), lambda i: (i, 0)),
    ],
    out_shape=[
        jax.ShapeDtypeStruct((NP, D), jnp.float32),
        jax.ShapeDtypeStruct((NP, D), jnp.float32),
    ],
)


def _tc3_body(sp_ref, z2_ref, dinv_ref, b2_ref, out_ref, ss_ref, h2_s):
    p = pl.program_id(0)
    i = pl.program_id(1)

    @pl.when(p == 0)
    def _():
        sp = sp_ref[...]
        dinv = dinv_ref[...]
        h = dinv * (sp[0] + sp[1]) + dinv * dinv * z2_ref[...] + b2_ref[...]
        rows = i * BLK + lax.broadcasted_iota(jnp.int32, (BLK, 1), 0)
        h = jnp.where(rows < N, h, 0.0)
        h2_s[pl.ds(i * BLK, BLK), :] = h

        @pl.when(i == 0)
        def _():
            ss_ref[...] = jnp.zeros_like(ss_ref)

        ss_ref[...] = ss_ref[...] + jnp.sum(h * h, axis=0, keepdims=True)

    @pl.when(p == 1)
    def _():
        denom = jnp.maximum(jnp.sqrt(ss_ref[...]), 1e-12)
        out_ref[...] = h2_s[pl.ds(i * BLK, BLK), :] / denom


_tc3 = pl.pallas_call(
    _tc3_body,
    grid=(2, NBLK),
    in_specs=[
        pl.BlockSpec((2, BLK, D), lambda p, i: (0, (1 - p) * i, 0)),
        pl.BlockSpec((BLK, D), lambda p, i: ((1 - p) * i, 0)),
        pl.BlockSpec((BLK, 1), lambda p, i: ((1 - p) * i, 0)),
        pl.BlockSpec((1, D), lambda p, i: (0, 0)),
    ],
    out_specs=[
        pl.BlockSpec((BLK, D), lambda p, i: (p * i, 0)),
        pl.BlockSpec((1, D), lambda p, i: (0, 0)),
    ],
    out_shape=[
        jax.ShapeDtypeStruct((NP, D), jnp.float32),
        jax.ShapeDtypeStruct((1, D), jnp.float32),
    ],
    scratch_shapes=[pltpu.VMEM((NP, D), jnp.float32)],
)


def kernel(node_features, edge_index, W1, b1, W2, b2):
    x = jnp.zeros((NP, D), jnp.float32).at[:N].set(node_features)
    ei = edge_index.astype(jnp.int32)
    pad = N + jnp.arange(EPAD - ei.shape[1], dtype=jnp.int32) % (NP - N)
    src_f = jnp.concatenate([ei[0], pad])
    dst_f = jnp.concatenate([ei[1], pad])
    idx = jnp.stack([src_f.reshape(NW, NCH, CH),
                     dst_f.reshape(NW, NCH, CH)], axis=2)
    zeros_deg = jnp.zeros((NP,), jnp.float32)
    zeros_tab = jnp.zeros((NP, D), jnp.float32)

    degp = _sc_deg(dst_f.reshape(NW, EPW), zeros_deg)
    dinv = _tc0(degp.reshape(NW, NP // D, D)).reshape(NP, 1)
    z1, p1 = _tc1(x, W1, dinv)
    s1 = _sc_agg(p1, idx, zeros_tab)
    z2, p2 = _tc2(s1, z1, dinv, b1.reshape(1, D), W2)
    s2 = _sc_agg(p2, idx, zeros_tab)
    out, _ = _tc3(s2, z2, dinv, b2.reshape(1, D))
    return out[:N]


# speedup vs baseline: 28.7276x; 1.0093x over previous
import jax
import jax.numpy as jnp
from jax import lax
from jax.experimental import pallas as pl
from jax.experimental.pallas import tpu as pltpu
from jax.experimental.pallas import tpu_sc as plsc

N = 10000
D = 128
NP = 10240
NC = 2
NS = 16
NW = NC * NS
CH = 80
EPW = 10240
NCH = EPW // CH
EPAD = NW * EPW
SENT = N + 64
RPS = NP // NS

_MESH = plsc.VectorSubcoreMesh(
    core_axis_name="c", subcore_axis_name="s", num_cores=NC, num_subcores=NS)


L = 16


def _sc_deg_body(dst_hbm, zeros_hbm, out_hbm, dst_v, accv):
    c = lax.axis_index("c")
    s = lax.axis_index("s")
    w = c * NS + s
    pltpu.sync_copy(dst_hbm.at[w], dst_v)
    pltpu.sync_copy(zeros_hbm, accv)
    ones = jnp.full((L,), 1.0, jnp.float32)

    def step(t, carry):
        plsc.addupdate_scatter(accv, [dst_v[pl.ds(t * L, L)]], ones)
        return carry

    lax.fori_loop(0, EPW // L, step, 0)
    pltpu.sync_copy(accv, out_hbm.at[pl.ds(w * NP, NP)])


_sc_deg = pl.kernel(
    _sc_deg_body,
    out_type=jax.ShapeDtypeStruct((NW * NP,), jnp.float32),
    mesh=_MESH,
    scratch_types=[
        pltpu.VMEM((EPW,), jnp.int32),
        pltpu.VMEM((NP,), jnp.float32),
    ],
    compiler_params=pltpu.CompilerParams(needs_layout_passes=False),
)


NBUF = 4
NIB = 8


def _sc_agg_body(tab_hbm, idx_hbm, zeros_hbm, out_hbm, idx_v, gb, acc, si, sg, ss):
    c = lax.axis_index("c")
    s = lax.axis_index("s")
    w = c * NS + s
    pltpu.sync_copy(zeros_hbm.at[pl.ds(s * RPS, RPS)], acc.at[pl.ds(s * RPS, RPS)])
    for q in range(NBUF):
        pltpu.async_copy(idx_hbm.at[w, q], idx_v.at[q], si.at[q])
    plsc.subcore_barrier()

    @pl.loop(0, NCH, step=NBUF)
    def _(g):
        for b in range(NBUF):
            j = g + b
            q = j % NIB

            @pl.when(j >= NBUF)
            def _():
                qq = (j - NBUF) % NIB
                pltpu.make_async_copy(gb.at[b], acc.at[idx_v.at[qq, 1]],
                                      ss.at[b]).wait()

            @pl.when(j + NBUF < NCH)
            def _():
                qn = (j + NBUF) % NIB
                pltpu.async_copy(idx_hbm.at[w, j + NBUF], idx_v.at[qn], si.at[qn])

            pltpu.make_async_copy(idx_hbm.at[w, j], idx_v.at[q], si.at[q]).wait()
            pltpu.async_copy(tab_hbm.at[idx_v.at[q, 0]], gb.at[b], sg.at[b])
        for b in range(NBUF):
            j = g + b
            q = j % NIB
            pltpu.make_async_copy(tab_hbm.at[idx_v.at[q, 0]], gb.at[b],
                                  sg.at[b]).wait()
            pltpu.async_copy(gb.at[b], acc.at[idx_v.at[q, 1]], ss.at[b], add=True)

    for b in range(NBUF):
        j = NCH - NBUF + b
        pltpu.make_async_copy(gb.at[b], acc.at[idx_v.at[j % NIB, 1]],
                              ss.at[b]).wait()

    plsc.subcore_barrier()
    pltpu.sync_copy(acc.at[pl.ds(s * RPS, RPS)], out_hbm.at[c, pl.ds(s * RPS, RPS)])


_sc_agg = pl.kernel(
    _sc_agg_body,
    out_type=jax.ShapeDtypeStruct((NC, NP, D), jnp.float32),
    mesh=_MESH,
    scratch_types=[
        pltpu.VMEM((NIB, 2, CH), jnp.int32),
        pltpu.VMEM((NBUF, CH, D), jnp.float32),
        pltpu.VMEM_SHARED((NP, D), jnp.float32),
        pltpu.SemaphoreType.DMA((NIB,)),
        pltpu.SemaphoreType.DMA((NBUF,)),
        pltpu.SemaphoreType.DMA((NBUF,)),
    ],
)


BLK = 512
NBLK = NP // BLK


def _tc0_body(degp_ref, dinv_ref):
    dinv_ref[...] = lax.rsqrt(jnp.sum(degp_ref[...], axis=0) + 1.0)


_tc0 = pl.pallas_call(
    _tc0_body,
    grid=(1,),
    in_specs=[pl.BlockSpec((NW, NP // D, D), lambda i: (0, 0, 0))],
    out_specs=pl.BlockSpec((NP // D, D), lambda i: (0, 0)),
    out_shape=jax.ShapeDtypeStruct((NP // D, D), jnp.float32),
)


def _tc1_body(x_ref, w1_ref, dinv_ref, z_ref, p_ref):
    dinv = dinv_ref[...]
    z = jnp.dot(x_ref[...], w1_ref[...],
                preferred_element_type=jnp.float32, precision=lax.Precision.HIGHEST)
    z_ref[...] = z
    p_ref[...] = z * dinv


_tc1 = pl.pallas_call(
    _tc1_body,
    grid=(NBLK,),
    in_specs=[
        pl.BlockSpec((BLK, D), lambda i: (i, 0)),
        pl.BlockSpec((D, D), lambda i: (0, 0)),
        pl.BlockSpec((BLK, 1), lambda i: (i, 0)),
    ],
    out_specs=[
        pl.BlockSpec((BLK, D), lambda i: (i, 0)),
        pl.BlockSpec((BLK, D), lambda i: (i, 0)),
    ],
    out_shape=[
        jax.ShapeDtypeStruct((NP, D), jnp.float32),
        jax.ShapeDtypeStruct((NP, D), jnp.float32),
    ],
)


def _tc2_body(sp_ref, z1_ref, dinv_ref, b1_ref, w2_ref, z2_ref, p2_ref):
    i = pl.program_id(0)
    sp = sp_ref[...]
    dinv = dinv_ref[...]
    h = dinv * (sp[0] + sp[1]) + dinv * dinv * z1_ref[...] + b1_ref[...]
    h = jnp.maximum(h, 0.0)
    rows = i * BLK + lax.broadcasted_iota(jnp.int32, (BLK, 1), 0)
    h = jnp.where(rows < N, h, 0.0)
    z2 = jnp.dot(h, w2_ref[...],
                 preferred_element_type=jnp.float32, precision=lax.Precision.HIGHEST)
    z2_ref[...] = z2
    p2_ref[...] = z2 * dinv


_tc2 = pl.pallas_call(
    _tc2_body,
    grid=(NBLK,),
    in_specs=[
        pl.BlockSpec((2, BLK, D), lambda i: (0, i, 0)),
        pl.BlockSpec((BLK, D), lambda i: (i, 0)),
        pl.BlockSpec((BLK, 1), lambda i: (i, 0)),
        pl.BlockSpec((1, D), lambda i: (0, 0)),
        pl.BlockSpec((D, D), lambda i: (0, 0)),
    ],
    out_specs=[
        pl.BlockSpec((BLK, D), lambda i: (i, 0)),
        pl.BlockSpec((BLK, D), lambda i: (i, 0)),
    ],
    out_shape=[
        jax.ShapeDtypeStruct((NP, D), jnp.float32),
        jax.ShapeDtypeStruct((NP, D), jnp.float32),
    ],
)


def _tc3_body(sp_ref, z2_ref, dinv_ref, b2_ref, out_ref, ss_ref, h2_s):
    p = pl.program_id(0)
    i = pl.program_id(1)

    @pl.when(p == 0)
    def _():
        sp = sp_ref[...]
        dinv = dinv_ref[...]
        h = dinv * (sp[0] + sp[1]) + dinv * dinv * z2_ref[...] + b2_ref[...]
        rows = i * BLK + lax.broadcasted_iota(jnp.int32, (BLK, 1), 0)
        h = jnp.where(rows < N, h, 0.0)
        h2_s[pl.ds(i * BLK, BLK), :] = h

        @pl.when(i == 0)
        def _():
            ss_ref[...] = jnp.zeros_like(ss_ref)

        ss_ref[...] = ss_ref[...] + jnp.sum(h * h, axis=0, keepdims=True)

    @pl.when(p == 1)
    def _():
        denom = jnp.maximum(jnp.sqrt(ss_ref[...]), 1e-12)
        out_ref[...] = h2_s[pl.ds(i * BLK, BLK), :] / denom


_tc3 = pl.pallas_call(
    _tc3_body,
    grid=(2, NBLK),
    in_specs=[
        pl.BlockSpec((2, BLK, D), lambda p, i: (0, (1 - p) * i, 0)),
        pl.BlockSpec((BLK, D), lambda p, i: ((1 - p) * i, 0)),
        pl.BlockSpec((BLK, 1), lambda p, i: ((1 - p) * i, 0)),
        pl.BlockSpec((1, D), lambda p, i: (0, 0)),
    ],
    out_specs=[
        pl.BlockSpec((BLK, D), lambda p, i: (p * i, 0)),
        pl.BlockSpec((1, D), lambda p, i: (0, 0)),
    ],
    out_shape=[
        jax.ShapeDtypeStruct((NP, D), jnp.float32),
        jax.ShapeDtypeStruct((1, D), jnp.float32),
    ],
    scratch_shapes=[pltpu.VMEM((NP, D), jnp.float32)],
)


def kernel(node_features, edge_index, W1, b1, W2, b2):
    x = jnp.zeros((NP, D), jnp.float32).at[:N].set(node_features)
    ei = edge_index.astype(jnp.int32)
    pad = N + jnp.arange(EPAD - ei.shape[1], dtype=jnp.int32) % (NP - N)
    src_f = jnp.concatenate([ei[0], pad])
    dst_f = jnp.concatenate([ei[1], pad])
    idx = jnp.stack([src_f.reshape(NW, NCH, CH),
                     dst_f.reshape(NW, NCH, CH)], axis=2)
    zeros_deg = jnp.zeros((NP,), jnp.float32)
    zeros_tab = jnp.zeros((NP, D), jnp.float32)

    degp = _sc_deg(dst_f.reshape(NW, EPW), zeros_deg)
    dinv = _tc0(degp.reshape(NW, NP // D, D)).reshape(NP, 1)
    z1, p1 = _tc1(x, W1, dinv)
    s1 = _sc_agg(p1, idx, zeros_tab)
    z2, p2 = _tc2(s1, z1, dinv, b1.reshape(1, D), W2)
    s2 = _sc_agg(p2, idx, zeros_tab)
    out, _ = _tc3(s2, z2, dinv, b2.reshape(1, D))
    return out[:N]
